# Initial kernel scaffold; baseline (speedup 1.0000x reference)
import functools

import jax
import jax.numpy as jnp
from jax import lax
from jax.experimental import pallas as pl
from jax.experimental.pallas import tpu as pltpu, tpu_sc as plsc

VOCAB = 1000000
EMB_DIM = 32
BATCH = 16384
HIST = 50

_info = plsc.get_sparse_core_info()
NC, NS = _info.num_cores, _info.num_subcores
NW = NC * NS

TOTAL = BATCH * HIST
PER_W = TOTAL // NW
CHUNK = 128
NCHUNK = PER_W // CHUNK


def _make_kernel():
    mesh = plsc.VectorSubcoreMesh(core_axis_name="c", subcore_axis_name="s")

    @functools.partial(
        pl.kernel,
        mesh=mesh,
        out_type=jax.ShapeDtypeStruct((NW, NCHUNK, CHUNK, EMB_DIM), jnp.float32),
        scratch_types=[
            pltpu.VMEM((NCHUNK, CHUNK), jnp.int32),
            pltpu.VMEM((CHUNK, EMB_DIM), jnp.float32),
            pltpu.SemaphoreType.DMA,
        ],
    )
    def gather_kernel(x_hbm, table_hbm, out_hbm, idx_v, rows_v, sem):
        wid = lax.axis_index("s") * NC + lax.axis_index("c")
        pltpu.sync_copy(x_hbm.at[wid], idx_v)

        def step(j, carry):
            pltpu.async_copy(table_hbm.at[idx_v.at[j]], rows_v, sem).wait()
            pltpu.sync_copy(rows_v, out_hbm.at[wid, j])
            return carry

        lax.fori_loop(0, NCHUNK, step, 0)

    return gather_kernel


_gather = _make_kernel()


def kernel(x, table):
    x_flat = x.reshape(NW, NCHUNK, CHUNK).astype(jnp.int32)
    out = _gather(x_flat, table)
    return out.reshape(BATCH, HIST, EMB_DIM)


# speedup vs baseline: 1.1870x; 1.1870x over previous
import functools

import jax
import jax.numpy as jnp
from jax import lax
from jax.experimental import pallas as pl
from jax.experimental.pallas import tpu as pltpu, tpu_sc as plsc

VOCAB = 1000000
EMB_DIM = 32
BATCH = 16384
HIST = 50

_info = plsc.get_sparse_core_info()
NC, NS = _info.num_cores, _info.num_subcores
NW = NC * NS

TOTAL = BATCH * HIST
PER_W = TOTAL // NW
CHUNK = 128
NCHUNK = PER_W // CHUNK


def _make_kernel():
    mesh = plsc.VectorSubcoreMesh(core_axis_name="c", subcore_axis_name="s")

    @functools.partial(
        pl.kernel,
        mesh=mesh,
        out_type=jax.ShapeDtypeStruct((NW, NCHUNK, CHUNK, EMB_DIM), jnp.float32),
        scratch_types=[
            pltpu.VMEM((NCHUNK, CHUNK), jnp.int32),
            pltpu.VMEM((CHUNK, EMB_DIM), jnp.float32),
            pltpu.SemaphoreType.DMA,
        ],
        compiler_params=pltpu.CompilerParams(use_tc_tiling_on_sc=False),
    )
    def gather_kernel(x_hbm, table_hbm, out_hbm, idx_v, rows_v, sem):
        wid = lax.axis_index("s") * NC + lax.axis_index("c")
        pltpu.sync_copy(x_hbm.at[wid], idx_v)

        def step(j, carry):
            pltpu.async_copy(table_hbm.at[idx_v.at[j]], rows_v, sem).wait()
            pltpu.sync_copy(rows_v, out_hbm.at[wid, j])
            return carry

        lax.fori_loop(0, NCHUNK, step, 0)

    return gather_kernel


_gather = _make_kernel()


def kernel(x, table):
    x_flat = x.reshape(NW, NCHUNK, CHUNK).astype(jnp.int32)
    out = _gather(x_flat, table)
    return out.reshape(BATCH, HIST, EMB_DIM)


# speedup vs baseline: 1.2777x; 1.0764x over previous
import functools

import jax
import jax.numpy as jnp
from jax import lax
from jax.experimental import pallas as pl
from jax.experimental.pallas import tpu as pltpu, tpu_sc as plsc

VOCAB = 1000000
EMB_DIM = 32
BATCH = 16384
HIST = 50

_info = plsc.get_sparse_core_info()
NC, NS = _info.num_cores, _info.num_subcores
NW = NC * NS

TOTAL = BATCH * HIST
PER_W = TOTAL // NW
CHUNK = 1280
NCHUNK = PER_W // CHUNK


def _make_kernel():
    mesh = plsc.VectorSubcoreMesh(core_axis_name="c", subcore_axis_name="s")

    @functools.partial(
        pl.kernel,
        mesh=mesh,
        out_type=jax.ShapeDtypeStruct((NW, NCHUNK, CHUNK, EMB_DIM), jnp.float32),
        scratch_types=[
            pltpu.VMEM((NCHUNK, CHUNK), jnp.int32),
            pltpu.VMEM((CHUNK, EMB_DIM), jnp.float32),
            pltpu.SemaphoreType.DMA,
        ],
        compiler_params=pltpu.CompilerParams(use_tc_tiling_on_sc=False),
    )
    def gather_kernel(x_hbm, table_hbm, out_hbm, idx_v, rows_v, sem):
        wid = lax.axis_index("s") * NC + lax.axis_index("c")
        pltpu.sync_copy(x_hbm.at[wid], idx_v)

        def step(j, carry):
            pltpu.async_copy(table_hbm.at[idx_v.at[j]], rows_v, sem).wait()
            pltpu.sync_copy(rows_v, out_hbm.at[wid, j])
            return carry

        lax.fori_loop(0, NCHUNK, step, 0)

    return gather_kernel


_gather = _make_kernel()


def kernel(x, table):
    x_flat = x.reshape(NW, NCHUNK, CHUNK).astype(jnp.int32)
    out = _gather(x_flat, table)
    return out.reshape(BATCH, HIST, EMB_DIM)


# speedup vs baseline: 1.3690x; 1.0715x over previous
import functools

import jax
import jax.numpy as jnp
from jax import lax
from jax.experimental import pallas as pl
from jax.experimental.pallas import tpu as pltpu, tpu_sc as plsc

VOCAB = 1000000
EMB_DIM = 32
BATCH = 16384
HIST = 50

_info = plsc.get_sparse_core_info()
NC, NS = _info.num_cores, _info.num_subcores
NW = NC * NS

TOTAL = BATCH * HIST
PER_W = TOTAL // NW
CHUNK = 800
NCHUNK = PER_W // CHUNK
NBUF = 4
NWAVE = NCHUNK // NBUF


def _make_kernel():
    mesh = plsc.VectorSubcoreMesh(core_axis_name="c", subcore_axis_name="s")

    @functools.partial(
        pl.kernel,
        mesh=mesh,
        out_type=jax.ShapeDtypeStruct((NW, NCHUNK, CHUNK, EMB_DIM), jnp.float32),
        scratch_types=[
            pltpu.VMEM((NCHUNK, CHUNK), jnp.int32),
            pltpu.VMEM((NBUF, CHUNK, EMB_DIM), jnp.float32),
            [pltpu.SemaphoreType.DMA] * NBUF,
            [pltpu.SemaphoreType.DMA] * NBUF,
        ],
        compiler_params=pltpu.CompilerParams(use_tc_tiling_on_sc=False),
    )
    def gather_kernel(x_hbm, table_hbm, out_hbm, idx_v, rows_v, gsems, osems):
        wid = lax.axis_index("s") * NC + lax.axis_index("c")
        pltpu.sync_copy(x_hbm.at[wid], idx_v)

        for b in range(NBUF):
            pltpu.async_copy(table_hbm.at[idx_v.at[b]], rows_v.at[b], gsems[b])

        def wave(i, carry):
            for b in range(NBUF):
                j = i * NBUF + b
                pltpu.make_async_copy(
                    table_hbm.at[idx_v.at[j]], rows_v.at[b], gsems[b]
                ).wait()
                pltpu.async_copy(rows_v.at[b], out_hbm.at[wid, j], osems[b])
            @pl.when(i < NWAVE - 1)
            def _():
                for b in range(NBUF):
                    j = i * NBUF + b
                    pltpu.make_async_copy(
                        rows_v.at[b], out_hbm.at[wid, j], osems[b]
                    ).wait()
                    pltpu.async_copy(
                        table_hbm.at[idx_v.at[j + NBUF]], rows_v.at[b], gsems[b]
                    )
            return carry

        lax.fori_loop(0, NWAVE, wave, 0)

        for b in range(NBUF):
            j = NCHUNK - NBUF + b
            pltpu.make_async_copy(
                rows_v.at[b], out_hbm.at[wid, j], osems[b]
            ).wait()

    return gather_kernel


_gather = _make_kernel()


def kernel(x, table):
    x_flat = x.reshape(NW, NCHUNK, CHUNK).astype(jnp.int32)
    out = _gather(x_flat, table)
    return out.reshape(BATCH, HIST, EMB_DIM)


# speedup vs baseline: 1.8043x; 1.3179x over previous
import functools

import jax
import jax.numpy as jnp
from jax import lax
from jax.experimental import pallas as pl
from jax.experimental.pallas import tpu as pltpu, tpu_sc as plsc

VOCAB = 1000000
EMB_DIM = 32
BATCH = 16384
HIST = 50

_info = plsc.get_sparse_core_info()
NC, NS = _info.num_cores, _info.num_subcores
NW = NC * NS

ROWS_W = BATCH // NW
BLK = 16
CHUNK = BLK * HIST
NBLK = ROWS_W // BLK
NBUF = 4
NWAVE = NBLK // NBUF


def _make_kernel():
    mesh = plsc.VectorSubcoreMesh(core_axis_name="c", subcore_axis_name="s")

    @functools.partial(
        pl.kernel,
        mesh=mesh,
        out_type=jax.ShapeDtypeStruct((BATCH, HIST, EMB_DIM), jnp.float32),
        scratch_types=[
            pltpu.VMEM((NBLK, CHUNK), jnp.int32),
            pltpu.VMEM((NBUF, CHUNK, EMB_DIM), jnp.float32),
            [pltpu.SemaphoreType.DMA] * NBUF,
            [pltpu.SemaphoreType.DMA] * NBUF,
            pltpu.SemaphoreType.DMA,
        ],
        compiler_params=pltpu.CompilerParams(use_tc_tiling_on_sc=False),
    )
    def gather_kernel(x_hbm, table_hbm, out_hbm, idx_v, rows_v, gsems, osems,
                      isem):
        wid = lax.axis_index("s") * NC + lax.axis_index("c")
        base = wid * ROWS_W
        for j in range(NBLK):
            pltpu.async_copy(
                x_hbm.at[pl.ds((base + j * BLK) * HIST, CHUNK)], idx_v.at[j],
                isem,
            )
        for j in range(NBLK):
            pltpu.make_async_copy(
                x_hbm.at[pl.ds((base + j * BLK) * HIST, CHUNK)], idx_v.at[j],
                isem,
            ).wait()

        def gather_blk(j, b):
            pltpu.async_copy(
                table_hbm.at[idx_v.at[j]], rows_v.at[b], gsems[b]
            )

        for b in range(NBUF):
            gather_blk(b, b)

        def store_blk(j, b):
            for r in range(BLK):
                pltpu.async_copy(
                    rows_v.at[b, pl.ds(r * HIST, HIST)],
                    out_hbm.at[base + j * BLK + r],
                    osems[b],
                )

        def store_blk_wait(j, b):
            for r in range(BLK):
                pltpu.make_async_copy(
                    rows_v.at[b, pl.ds(r * HIST, HIST)],
                    out_hbm.at[base + j * BLK + r],
                    osems[b],
                ).wait()

        def wave(i, carry):
            for b in range(NBUF):
                j = i * NBUF + b
                pltpu.make_async_copy(
                    table_hbm.at[idx_v.at[j]], rows_v.at[b], gsems[b]
                ).wait()
                store_blk(j, b)
            @pl.when(i < NWAVE - 1)
            def _():
                for b in range(NBUF):
                    j = i * NBUF + b
                    store_blk_wait(j, b)
                    gather_blk(j + NBUF, b)
            return carry

        lax.fori_loop(0, NWAVE, wave, 0)

        for b in range(NBUF):
            store_blk_wait(NBLK - NBUF + b, b)

    return gather_kernel


_gather = _make_kernel()


def kernel(x, table):
    x_flat = x.reshape(BATCH * HIST).astype(jnp.int32)
    return _gather(x_flat, table)
